# baseline (speedup 1.0000x reference)
import jax
import jax.numpy as jnp
from jax.experimental import pallas as pl


def _dot(a, b):
    return jax.lax.dot_general(
        a, b, (((1,), (0,)), ((), ())),
        precision=jax.lax.Precision.DEFAULT,
        preferred_element_type=jnp.float32,
    )


def _s1_body(x_ref, w1_ref, s1_ref):
    s1_ref[...] = _dot(x_ref[...], w1_ref[...])


def _layer1_body(adj_ref, s1_ref, b1_ref, w2_ref, s2_ref):
    h = _dot(adj_ref[...], s1_ref[...])
    h = jnp.maximum(h + b1_ref[...], 0.0)
    s2_ref[...] = _dot(h, w2_ref[...])


def _layer2_body(adj_ref, s2_ref, b2_ref, out_ref):
    o = _dot(adj_ref[...], s2_ref[...])
    o = o + b2_ref[...]
    m = jnp.max(o, axis=1, keepdims=True)
    lse = m + jnp.log(jnp.sum(jnp.exp(o - m), axis=1, keepdims=True))
    out_ref[...] = o - lse


def kernel(x, adj, layer_dropout, stage1_flag, W1, b1, W2, b2):
    n, nfeat = x.shape
    nhid = W1.shape[1]
    nclass = W2.shape[1]

    bm_s1 = 2000
    s1 = pl.pallas_call(
        _s1_body,
        grid=(n // bm_s1,),
        in_specs=[
            pl.BlockSpec((bm_s1, nfeat), lambda i: (i, 0)),
            pl.BlockSpec((nfeat, nhid), lambda i: (0, 0)),
        ],
        out_specs=pl.BlockSpec((bm_s1, nhid), lambda i: (i, 0)),
        out_shape=jax.ShapeDtypeStruct((n, nhid), jnp.float32),
    )(x, W1)

    b1_2d = b1.reshape(1, nhid)
    b2_2d = b2.reshape(1, nclass)

    bm = 200
    s2 = pl.pallas_call(
        _layer1_body,
        grid=(n // bm,),
        in_specs=[
            pl.BlockSpec((bm, n), lambda i: (i, 0)),
            pl.BlockSpec((n, nhid), lambda i: (0, 0)),
            pl.BlockSpec((1, nhid), lambda i: (0, 0)),
            pl.BlockSpec((nhid, nclass), lambda i: (0, 0)),
        ],
        out_specs=pl.BlockSpec((bm, nclass), lambda i: (i, 0)),
        out_shape=jax.ShapeDtypeStruct((n, nclass), jnp.float32),
    )(adj, s1, b1_2d, W2)

    logp = pl.pallas_call(
        _layer2_body,
        grid=(n // bm,),
        in_specs=[
            pl.BlockSpec((bm, n), lambda i: (i, 0)),
            pl.BlockSpec((n, nclass), lambda i: (0, 0)),
            pl.BlockSpec((1, nclass), lambda i: (0, 0)),
        ],
        out_specs=pl.BlockSpec((bm, nclass), lambda i: (i, 0)),
        out_shape=jax.ShapeDtypeStruct((n, nclass), jnp.float32),
    )(adj, s2, b2_2d)

    node_lastlayer = jnp.ones((n, 1), dtype=jnp.float32)
    return (logp, node_lastlayer)


# speedup vs baseline: 1.0571x; 1.0571x over previous
import jax
import jax.numpy as jnp
from jax.experimental import pallas as pl
from jax.experimental.pallas import tpu as pltpu


def _dot(a, b):
    return jax.lax.dot_general(
        a, b, (((1,), (0,)), ((), ())),
        precision=jax.lax.Precision.DEFAULT,
        preferred_element_type=jnp.float32,
    )


def _body(x_ref, w1_ref, b1_ref, w2_ref, b2_ref, adj_ref, out_ref,
          s1_ref, s2_ref, *, bm, tiles):
    s = pl.program_id(0)

    @pl.when(s == 0)
    def _():
        s1_ref[...] = _dot(x_ref[...], w1_ref[...])

    @pl.when(jnp.logical_and(s >= 1, s <= tiles))
    def _():
        h = _dot(adj_ref[...], s1_ref[...])
        h = jnp.maximum(h + b1_ref[...], 0.0)
        s2_ref[pl.ds((s - 1) * bm, bm), :] = _dot(h, w2_ref[...])

    @pl.when(s > tiles)
    def _():
        o = _dot(adj_ref[...], s2_ref[...]) + b2_ref[...]
        m = jnp.max(o, axis=1, keepdims=True)
        lse = m + jnp.log(jnp.sum(jnp.exp(o - m), axis=1, keepdims=True))
        out_ref[...] = o - lse


def kernel(x, adj, layer_dropout, stage1_flag, W1, b1, W2, b2):
    n, nfeat = x.shape
    nhid = W1.shape[1]
    nclass = W2.shape[1]

    bm = 400
    tiles = n // bm
    t1 = tiles + 1

    def adj_map(s):
        return (jnp.where(s < t1, jnp.maximum(s - 1, 0), s - t1), 0)

    def out_map(s):
        return (jnp.where(s < t1, 0, s - t1), 0)

    import functools
    body = functools.partial(_body, bm=bm, tiles=tiles)

    logp = pl.pallas_call(
        body,
        grid=(1 + 2 * tiles,),
        in_specs=[
            pl.BlockSpec((n, nfeat), lambda s: (0, 0)),
            pl.BlockSpec((nfeat, nhid), lambda s: (0, 0)),
            pl.BlockSpec((1, nhid), lambda s: (0, 0)),
            pl.BlockSpec((nhid, nclass), lambda s: (0, 0)),
            pl.BlockSpec((1, nclass), lambda s: (0, 0)),
            pl.BlockSpec((bm, n), adj_map),
        ],
        out_specs=pl.BlockSpec((bm, nclass), out_map),
        out_shape=jax.ShapeDtypeStruct((n, nclass), jnp.float32),
        scratch_shapes=[
            pltpu.VMEM((n, nhid), jnp.float32),
            pltpu.VMEM((n, nclass), jnp.float32),
        ],
    )(x, W1, b1.reshape(1, nhid), W2, b2.reshape(1, nclass), adj)

    node_lastlayer = jnp.ones((n, 1), dtype=jnp.float32)
    return (logp, node_lastlayer)


# speedup vs baseline: 1.0587x; 1.0016x over previous
import jax
import jax.numpy as jnp
from jax.experimental import pallas as pl
from jax.experimental.pallas import tpu as pltpu


def _dot(a, b):
    return jax.lax.dot_general(
        a, b, (((1,), (0,)), ((), ())),
        precision=jax.lax.Precision.DEFAULT,
        preferred_element_type=jnp.float32,
    )


def _body(x_ref, w1_ref, b1_ref, w2_ref, b2_ref, adj_ref, out_ref,
          s1_ref, s2_ref, *, bm, tiles):
    s = pl.program_id(0)

    @pl.when(s == 0)
    def _():
        s1_ref[...] = _dot(x_ref[...], w1_ref[...]).astype(jnp.bfloat16)

    @pl.when(jnp.logical_and(s >= 1, s <= tiles))
    def _():
        a = adj_ref[...].astype(jnp.bfloat16)
        h = _dot(a, s1_ref[...])
        h = jnp.maximum(h + b1_ref[...], 0.0).astype(jnp.bfloat16)
        s2_ref[pl.ds((s - 1) * bm, bm), :] = _dot(h, w2_ref[...]).astype(
            jnp.bfloat16)

    @pl.when(s > tiles)
    def _():
        a = adj_ref[...].astype(jnp.bfloat16)
        o = _dot(a, s2_ref[...]) + b2_ref[...]
        m = jnp.max(o, axis=1, keepdims=True)
        lse = m + jnp.log(jnp.sum(jnp.exp(o - m), axis=1, keepdims=True))
        out_ref[...] = o - lse


def kernel(x, adj, layer_dropout, stage1_flag, W1, b1, W2, b2):
    n, nfeat = x.shape
    nhid = W1.shape[1]
    nclass = W2.shape[1]

    bm = 400
    tiles = n // bm
    t1 = tiles + 1

    def adj_map(s):
        return (jnp.where(s < t1, jnp.maximum(s - 1, 0), s - t1), 0)

    def out_map(s):
        return (jnp.where(s < t1, 0, s - t1), 0)

    import functools
    body = functools.partial(_body, bm=bm, tiles=tiles)

    logp = pl.pallas_call(
        body,
        grid=(1 + 2 * tiles,),
        in_specs=[
            pl.BlockSpec((n, nfeat), lambda s: (0, 0)),
            pl.BlockSpec((nfeat, nhid), lambda s: (0, 0)),
            pl.BlockSpec((1, nhid), lambda s: (0, 0)),
            pl.BlockSpec((nhid, nclass), lambda s: (0, 0)),
            pl.BlockSpec((1, nclass), lambda s: (0, 0)),
            pl.BlockSpec((bm, n), adj_map),
        ],
        out_specs=pl.BlockSpec((bm, nclass), out_map),
        out_shape=jax.ShapeDtypeStruct((n, nclass), jnp.float32),
        scratch_shapes=[
            pltpu.VMEM((n, nhid), jnp.bfloat16),
            pltpu.VMEM((n, nclass), jnp.bfloat16),
        ],
    )(x, W1, b1.reshape(1, nhid), W2.astype(jnp.bfloat16),
      b2.reshape(1, nclass), adj)

    node_lastlayer = jnp.ones((n, 1), dtype=jnp.float32)
    return (logp, node_lastlayer)
